# baseline (device time: 25326 ns/iter reference)
import jax
import jax.numpy as jnp
from jax import lax
from jax.experimental import pallas as pl
from jax.experimental.pallas import tpu as pltpu

N_DEV = 4
B, Sq, SKV_LOC, Hq, Dh = 2, 128, 128, 4, 64
D_MODEL = 512
D_QK = Hq * Dh


def kernel(x, Wq, K_ext, V_ext, Wo):
    def body(x_ref, wq_ref, k_ref, v_ref, wo_ref, out_ref,
             o_all, ms_all, send_sems, recv_sems):
        my = lax.axis_index("i")

        xm = x_ref[...].reshape(B * Sq, D_MODEL)
        q = jnp.dot(xm, wq_ref[...], preferred_element_type=jnp.float32)

        qb = lax.broadcasted_iota(jnp.int32, (Sq, SKV_LOC), 0) // 64
        kb = lax.broadcasted_iota(jnp.int32, (Sq, SKV_LOC), 1) // 64 + 2 * my
        mask = (qb == kb) | ((kb % 4) == (qb % 4))

        for b in range(B):
            for h in range(Hq):
                qbh = q[b * Sq:(b + 1) * Sq, h * Dh:(h + 1) * Dh]
                kbh = k_ref[b, :, h, :]
                s = lax.dot_general(
                    qbh, kbh, (((1,), (1,)), ((), ())),
                    preferred_element_type=jnp.float32) * 0.125
                s = jnp.where(mask, s, -1e9)
                m = jnp.max(s, axis=1)
                w = jnp.exp(s - m[:, None])
                ssum = jnp.sum(w, axis=1)
                o = jnp.dot(w, v_ref[b, :, h, :],
                            preferred_element_type=jnp.float32)
                o_all[0, b, h] = o
                ms_all[0, b, h, 0] = m
                ms_all[0, b, h, 1] = ssum

        sends = []
        for j in range(1, N_DEV):
            tgt = (my + j) % N_DEV
            dst_slot = N_DEV - j
            for t, buf in enumerate((o_all, ms_all)):
                rdma = pltpu.make_async_remote_copy(
                    src_ref=buf.at[0],
                    dst_ref=buf.at[dst_slot],
                    send_sem=send_sems.at[j - 1, t],
                    recv_sem=recv_sems.at[dst_slot, t],
                    device_id=(tgt,),
                    device_id_type=pl.DeviceIdType.MESH,
                )
                rdma.start()
                sends.append(rdma)

        for r in range(1, N_DEV):
            for t, buf in enumerate((o_all, ms_all)):
                recv = pltpu.make_async_remote_copy(
                    src_ref=buf.at[0],
                    dst_ref=buf.at[r],
                    send_sem=send_sems.at[0, t],
                    recv_sem=recv_sems.at[r, t],
                    device_id=(my,),
                    device_id_type=pl.DeviceIdType.MESH,
                )
                recv.wait_recv()

        for b in range(B):
            out_b = jnp.zeros((Sq, D_MODEL), dtype=jnp.float32)
            for h in range(Hq):
                m_r = jnp.stack(
                    [ms_all[r, b, h, 0] for r in range(N_DEV)])
                s_r = jnp.stack(
                    [ms_all[r, b, h, 1] for r in range(N_DEV)])
                m_g = jnp.max(m_r, axis=0)
                scale = jnp.exp(m_r - m_g[None, :])
                denom = jnp.sum(s_r * scale, axis=0)
                acc = jnp.zeros((Sq, Dh), dtype=jnp.float32)
                for r in range(N_DEV):
                    acc = acc + o_all[r, b, h] * scale[r][:, None]
                ctx = acc / denom[:, None]
                out_b = out_b + jnp.dot(
                    ctx, wo_ref[h * Dh:(h + 1) * Dh, :],
                    preferred_element_type=jnp.float32)
            out_ref[b] = out_b

        for rdma in sends:
            rdma.wait_send()

    return pl.pallas_call(
        body,
        out_shape=jax.ShapeDtypeStruct((B, Sq, D_MODEL), jnp.float32),
        in_specs=[pl.BlockSpec(memory_space=pltpu.VMEM)] * 5,
        out_specs=pl.BlockSpec(memory_space=pltpu.VMEM),
        scratch_shapes=[
            pltpu.VMEM((N_DEV, B, Hq, Sq, Dh), jnp.float32),
            pltpu.VMEM((N_DEV, B, Hq, 2, Sq), jnp.float32),
            pltpu.SemaphoreType.DMA((N_DEV - 1, 2)),
            pltpu.SemaphoreType.DMA((N_DEV, 2)),
        ],
    )(x, Wq, K_ext, V_ext, Wo)


# device time: 18686 ns/iter; 1.3553x vs baseline; 1.3553x over previous
import jax
import jax.numpy as jnp
from jax import lax
from jax.experimental import pallas as pl
from jax.experimental.pallas import tpu as pltpu

N_DEV = 4
B, Sq, SKV_LOC, Hq, Dh = 2, 128, 128, 4, 64
D_MODEL = 512
D_QK = Hq * Dh

EVENS = (0, 2)
SEND_TARGETS = {0: (1, 2, 3), 2: (3, 0, 1)}


def kernel(x, Wq, K_ext, V_ext, Wo):
    K2 = K_ext.reshape(B, SKV_LOC, D_QK)
    V2 = V_ext.reshape(B, SKV_LOC, D_QK)

    def body(x_ref, wq_ref, k_ref, v_ref, wo_ref, out_ref,
             o_all, ms_all, send_sems, recv_sems, done_sem):
        my = lax.axis_index("i")

        barrier = pltpu.get_barrier_semaphore()
        for tgt in EVENS:
            @pl.when(my != tgt)
            def _():
                pl.semaphore_signal(barrier, inc=1, device_id=(tgt,),
                                    device_id_type=pl.DeviceIdType.MESH)

        def compute_partial(slot, dev):
            xm = x_ref[...].reshape(B * Sq, D_MODEL)
            q = jnp.dot(xm, wq_ref[...], preferred_element_type=jnp.float32)
            qb = lax.broadcasted_iota(jnp.int32, (Sq, SKV_LOC), 0) // 64
            kb = lax.broadcasted_iota(jnp.int32, (Sq, SKV_LOC), 1) // 64 + 2 * dev
            mask = (qb == kb) | ((kb % 4) == (qb % 4))
            for b in range(B):
                for h in range(Hq):
                    qbh = q[b * Sq:(b + 1) * Sq, h * Dh:(h + 1) * Dh]
                    kbh = k_ref[b, :, h * Dh:(h + 1) * Dh]
                    s = lax.dot_general(
                        qbh, kbh, (((1,), (1,)), ((), ())),
                        preferred_element_type=jnp.float32) * 0.125
                    s = jnp.where(mask, s, -1e9)
                    m = jnp.max(s, axis=1)
                    w = jnp.exp(s - m[:, None])
                    ssum = jnp.sum(w, axis=1)
                    o = jnp.dot(w, v_ref[b, :, h * Dh:(h + 1) * Dh],
                                preferred_element_type=jnp.float32)
                    o_all[slot, b, :, h * Dh:(h + 1) * Dh] = o
                    ms_all[slot, b, h, 0] = m
                    ms_all[slot, b, h, 1] = ssum

        def make_rdmas(slot, dev):
            rdmas = []
            for j, tgt in enumerate(SEND_TARGETS[dev]):
                for t, buf in enumerate((o_all, ms_all)):
                    rdmas.append(pltpu.make_async_remote_copy(
                        src_ref=buf.at[slot],
                        dst_ref=buf.at[slot],
                        send_sem=send_sems.at[j, t],
                        recv_sem=recv_sems.at[slot, t],
                        device_id=(tgt,),
                        device_id_type=pl.DeviceIdType.MESH,
                    ))
            return rdmas

        for slot, dev in enumerate(EVENS):
            @pl.when(my == dev)
            def _(slot=slot, dev=dev):
                pl.semaphore_wait(barrier, 3)
                compute_partial(slot, dev)
                for rdma in make_rdmas(slot, dev):
                    rdma.start()

        for slot, dev in enumerate(EVENS):
            @pl.when(my != dev)
            def _(slot=slot):
                for t, buf in enumerate((o_all, ms_all)):
                    recv = pltpu.make_async_remote_copy(
                        src_ref=buf.at[slot],
                        dst_ref=buf.at[slot],
                        send_sem=send_sems.at[0, t],
                        recv_sem=recv_sems.at[slot, t],
                        device_id=(0,),
                        device_id_type=pl.DeviceIdType.MESH,
                    )
                    recv.wait_recv()

        m0 = ms_all[0, :, :, 0]
        m1 = ms_all[1, :, :, 0]
        s0 = ms_all[0, :, :, 1]
        s1 = ms_all[1, :, :, 1]
        m_g = jnp.maximum(m0, m1)
        e0 = jnp.exp(m0 - m_g)
        e1 = jnp.exp(m1 - m_g)
        denom = s0 * e0 + s1 * e1
        c0 = e0 / denom
        c1 = e1 / denom

        def expand(c):
            return jnp.concatenate(
                [jnp.broadcast_to(c[:, h, :, None], (B, Sq, Dh))
                 for h in range(Hq)], axis=-1)

        ctx = o_all[0] * expand(c0) + o_all[1] * expand(c1)
        out = jnp.dot(ctx.reshape(B * Sq, D_QK), wo_ref[...],
                      preferred_element_type=jnp.float32)
        out_ref[...] = out.reshape(B, Sq, D_MODEL)

        @pl.when((my == 1) | (my == 3))
        def _():
            for tgt in EVENS:
                pl.semaphore_signal(done_sem, inc=1, device_id=(tgt,),
                                    device_id_type=pl.DeviceIdType.MESH)

        for slot, dev in enumerate(EVENS):
            @pl.when(my == dev)
            def _(slot=slot, dev=dev):
                for rdma in make_rdmas(slot, dev):
                    rdma.wait_send()
                pl.semaphore_wait(done_sem, 2)

    return pl.pallas_call(
        body,
        out_shape=jax.ShapeDtypeStruct((B, Sq, D_MODEL), jnp.float32),
        in_specs=[pl.BlockSpec(memory_space=pltpu.VMEM)] * 5,
        out_specs=pl.BlockSpec(memory_space=pltpu.VMEM),
        scratch_shapes=[
            pltpu.VMEM((2, B, Sq, D_QK), jnp.float32),
            pltpu.VMEM((2, B, Hq, 2, Sq), jnp.float32),
            pltpu.SemaphoreType.DMA((3, 2)),
            pltpu.SemaphoreType.DMA((2, 2)),
            pltpu.SemaphoreType.REGULAR,
        ],
        compiler_params=pltpu.CompilerParams(collective_id=0),
    )(x, Wq, K2, V2, Wo)


# device time: 17180 ns/iter; 1.4742x vs baseline; 1.0877x over previous
import jax
import jax.numpy as jnp
from jax import lax
from jax.experimental import pallas as pl
from jax.experimental.pallas import tpu as pltpu

N_DEV = 4
B, Sq, SKV_LOC, Hq, Dh = 2, 128, 128, 4, 64
D_MODEL = 512
D_QK = Hq * Dh

EVENS = (0, 2)
SEND_TARGETS = {0: (1, 2, 3), 2: (3, 0, 1)}


def kernel(x, Wq, K_ext, V_ext, Wo):
    KT = jnp.transpose(K_ext, (0, 2, 3, 1))
    VT = jnp.transpose(V_ext, (0, 2, 3, 1))

    def body(x_ref, wq_ref, k_ref, v_ref, wo_ref, out_ref,
             o_all, ms_all, send_sems, recv_sems, done_sem):
        my = lax.axis_index("i")

        barrier = pltpu.get_barrier_semaphore()
        for tgt in EVENS:
            @pl.when(my != tgt)
            def _():
                pl.semaphore_signal(barrier, inc=1, device_id=(tgt,),
                                    device_id_type=pl.DeviceIdType.MESH)

        def compute_partial(slot, dev):
            xm = x_ref[...].reshape(B * Sq, D_MODEL)
            q = jnp.dot(xm, wq_ref[...], preferred_element_type=jnp.float32)
            qb = lax.broadcasted_iota(jnp.int32, (Sq, SKV_LOC), 0) // 64
            kb = lax.broadcasted_iota(jnp.int32, (Sq, SKV_LOC), 1) // 64 + 2 * dev
            mask = (qb == kb) | ((kb % 4) == (qb % 4))
            for b in range(B):
                for h in range(Hq):
                    qbh = q[b * Sq:(b + 1) * Sq, h * Dh:(h + 1) * Dh]
                    s = jnp.dot(qbh, k_ref[b, h],
                                preferred_element_type=jnp.float32) * 0.125
                    s = jnp.where(mask, s, -1e9)
                    m = jnp.max(s, axis=1)
                    w = jnp.exp(s - m[:, None])
                    ssum = jnp.sum(w, axis=1)
                    o = lax.dot_general(
                        w, v_ref[b, h], (((1,), (1,)), ((), ())),
                        preferred_element_type=jnp.float32)
                    o_all[slot, b, :, h * Dh:(h + 1) * Dh] = o.astype(
                        jnp.bfloat16)
                    ms_all[slot, b, h, 0] = m
                    ms_all[slot, b, h, 1] = ssum

        def make_rdmas(slot, dev):
            rdmas = []
            for j, tgt in enumerate(SEND_TARGETS[dev]):
                for t, buf in enumerate((o_all, ms_all)):
                    rdmas.append(pltpu.make_async_remote_copy(
                        src_ref=buf.at[slot],
                        dst_ref=buf.at[slot],
                        send_sem=send_sems.at[j, t],
                        recv_sem=recv_sems.at[slot, t],
                        device_id=(tgt,),
                        device_id_type=pl.DeviceIdType.MESH,
                    ))
            return rdmas

        for slot, dev in enumerate(EVENS):
            @pl.when(my == dev)
            def _(slot=slot, dev=dev):
                pl.semaphore_wait(barrier, 3)
                compute_partial(slot, dev)
                for rdma in make_rdmas(slot, dev):
                    rdma.start()

        for slot, dev in enumerate(EVENS):
            @pl.when(my != dev)
            def _(slot=slot):
                for t, buf in enumerate((o_all, ms_all)):
                    recv = pltpu.make_async_remote_copy(
                        src_ref=buf.at[slot],
                        dst_ref=buf.at[slot],
                        send_sem=send_sems.at[0, t],
                        recv_sem=recv_sems.at[slot, t],
                        device_id=(0,),
                        device_id_type=pl.DeviceIdType.MESH,
                    )
                    recv.wait_recv()

        m0 = ms_all[0, :, :, 0]
        m1 = ms_all[1, :, :, 0]
        s0 = ms_all[0, :, :, 1]
        s1 = ms_all[1, :, :, 1]
        m_g = jnp.maximum(m0, m1)
        e0 = jnp.exp(m0 - m_g)
        e1 = jnp.exp(m1 - m_g)
        denom = s0 * e0 + s1 * e1
        c0 = e0 / denom
        c1 = e1 / denom

        def expand(c):
            return jnp.concatenate(
                [jnp.broadcast_to(c[:, h, :, None], (B, Sq, Dh))
                 for h in range(Hq)], axis=-1)

        ctx = (o_all[0].astype(jnp.float32) * expand(c0)
               + o_all[1].astype(jnp.float32) * expand(c1))
        out = jnp.dot(ctx.reshape(B * Sq, D_QK), wo_ref[...],
                      preferred_element_type=jnp.float32)
        out_ref[...] = out.reshape(B, Sq, D_MODEL)

        @pl.when((my == 1) | (my == 3))
        def _():
            for tgt in EVENS:
                pl.semaphore_signal(done_sem, inc=1, device_id=(tgt,),
                                    device_id_type=pl.DeviceIdType.MESH)

        for slot, dev in enumerate(EVENS):
            @pl.when(my == dev)
            def _(slot=slot, dev=dev):
                for rdma in make_rdmas(slot, dev):
                    rdma.wait_send()
                pl.semaphore_wait(done_sem, 2)

    return pl.pallas_call(
        body,
        out_shape=jax.ShapeDtypeStruct((B, Sq, D_MODEL), jnp.float32),
        in_specs=[pl.BlockSpec(memory_space=pltpu.VMEM)] * 5,
        out_specs=pl.BlockSpec(memory_space=pltpu.VMEM),
        scratch_shapes=[
            pltpu.VMEM((2, B, Sq, D_QK), jnp.bfloat16),
            pltpu.VMEM((2, B, Hq, 2, Sq), jnp.float32),
            pltpu.SemaphoreType.DMA((3, 2)),
            pltpu.SemaphoreType.DMA((2, 2)),
            pltpu.SemaphoreType.REGULAR,
        ],
        compiler_params=pltpu.CompilerParams(collective_id=0),
    )(x, Wq, KT, VT, Wo)


# device time: 16482 ns/iter; 1.5366x vs baseline; 1.0423x over previous
import jax
import jax.numpy as jnp
from jax import lax
from jax.experimental import pallas as pl
from jax.experimental.pallas import tpu as pltpu

N_DEV = 4
B, Sq, SKV_LOC, Hq, Dh = 2, 128, 128, 4, 64
D_MODEL = 512
D_QK = Hq * Dh

EVENS = (0, 2)
SEND_TARGETS = {0: (1, 2, 3), 2: (3, 0, 1)}


def kernel(x, Wq, K_ext, V_ext, Wo):
    KT = jnp.transpose(K_ext, (0, 2, 3, 1))
    VT = jnp.transpose(V_ext, (0, 2, 3, 1))

    def body(x_hbm, wq_hbm, kt_hbm, vt_hbm, wo_hbm, out_hbm,
             x_v, wq_v, kt_v, vt_v, wo_v, out_v,
             o_all, ms_all, in_sems, send_sems, recv_sems, done_sem):
        my = lax.axis_index("i")

        wo_dma = pltpu.make_async_copy(wo_hbm, wo_v, in_sems.at[4])
        wo_dma.start()

        barrier = pltpu.get_barrier_semaphore()
        for tgt in EVENS:
            @pl.when(my != tgt)
            def _():
                pl.semaphore_signal(barrier, inc=1, device_id=(tgt,),
                                    device_id_type=pl.DeviceIdType.MESH)

        def compute_partial(slot, dev):
            q = jnp.dot(x_v[...].reshape(B * Sq, D_MODEL), wq_v[...],
                        preferred_element_type=jnp.float32)
            qb = lax.broadcasted_iota(jnp.int32, (Sq, SKV_LOC), 0) // 64
            kb = lax.broadcasted_iota(jnp.int32, (Sq, SKV_LOC), 1) // 64 + 2 * dev
            mask = (qb == kb) | ((kb % 4) == (qb % 4))
            for b in range(B):
                for h in range(Hq):
                    qbh = q[b * Sq:(b + 1) * Sq, h * Dh:(h + 1) * Dh]
                    s = jnp.dot(qbh, kt_v[b, h],
                                preferred_element_type=jnp.float32) * 0.125
                    s = jnp.where(mask, s, -1e9)
                    m = jnp.max(s, axis=1)
                    w = jnp.exp(s - m[:, None])
                    ssum = jnp.sum(w, axis=1)
                    o = lax.dot_general(
                        w, vt_v[b, h], (((1,), (1,)), ((), ())),
                        preferred_element_type=jnp.float32)
                    o_all[slot, b, :, h * Dh:(h + 1) * Dh] = o.astype(
                        jnp.bfloat16)
                    ms_all[slot, b, h, 0] = m
                    ms_all[slot, b, h, 1] = ssum

        def make_rdmas(slot, dev):
            rdmas = []
            for j, tgt in enumerate(SEND_TARGETS[dev]):
                for t, buf in enumerate((o_all, ms_all)):
                    rdmas.append(pltpu.make_async_remote_copy(
                        src_ref=buf.at[slot],
                        dst_ref=buf.at[slot],
                        send_sem=send_sems.at[j, t],
                        recv_sem=recv_sems.at[slot, t],
                        device_id=(tgt,),
                        device_id_type=pl.DeviceIdType.MESH,
                    ))
            return rdmas

        for slot, dev in enumerate(EVENS):
            @pl.when(my == dev)
            def _(slot=slot, dev=dev):
                dmas = [pltpu.make_async_copy(x_hbm, x_v, in_sems.at[0]),
                        pltpu.make_async_copy(wq_hbm, wq_v, in_sems.at[1]),
                        pltpu.make_async_copy(kt_hbm, kt_v, in_sems.at[2]),
                        pltpu.make_async_copy(vt_hbm, vt_v, in_sems.at[3])]
                for d in dmas:
                    d.start()
                pl.semaphore_wait(barrier, 3)
                for d in dmas:
                    d.wait()
                compute_partial(slot, dev)
                for rdma in make_rdmas(slot, dev):
                    rdma.start()

        for slot, dev in enumerate(EVENS):
            @pl.when(my != dev)
            def _(slot=slot):
                for t, buf in enumerate((o_all, ms_all)):
                    recv = pltpu.make_async_remote_copy(
                        src_ref=buf.at[slot],
                        dst_ref=buf.at[slot],
                        send_sem=send_sems.at[0, t],
                        recv_sem=recv_sems.at[slot, t],
                        device_id=(0,),
                        device_id_type=pl.DeviceIdType.MESH,
                    )
                    recv.wait_recv()

        m0 = ms_all[0, :, :, 0]
        m1 = ms_all[1, :, :, 0]
        s0 = ms_all[0, :, :, 1]
        s1 = ms_all[1, :, :, 1]
        m_g = jnp.maximum(m0, m1)
        e0 = jnp.exp(m0 - m_g)
        e1 = jnp.exp(m1 - m_g)
        denom = s0 * e0 + s1 * e1
        c0 = e0 / denom
        c1 = e1 / denom

        def expand(c):
            return jnp.concatenate(
                [jnp.broadcast_to(c[:, h, :, None], (B, Sq, Dh))
                 for h in range(Hq)], axis=-1)

        ctx = (o_all[0].astype(jnp.float32) * expand(c0)
               + o_all[1].astype(jnp.float32) * expand(c1))
        wo_dma.wait()
        out = jnp.dot(ctx.reshape(B * Sq, D_QK), wo_v[...],
                      preferred_element_type=jnp.float32)
        out_v[...] = out.reshape(B, Sq, D_MODEL)
        out_dma = pltpu.make_async_copy(out_v, out_hbm, in_sems.at[5])
        out_dma.start()

        @pl.when((my == 1) | (my == 3))
        def _():
            for tgt in EVENS:
                pl.semaphore_signal(done_sem, inc=1, device_id=(tgt,),
                                    device_id_type=pl.DeviceIdType.MESH)

        for slot, dev in enumerate(EVENS):
            @pl.when(my == dev)
            def _(slot=slot, dev=dev):
                for rdma in make_rdmas(slot, dev):
                    rdma.wait_send()
                pl.semaphore_wait(done_sem, 2)

        out_dma.wait()

    return pl.pallas_call(
        body,
        out_shape=jax.ShapeDtypeStruct((B, Sq, D_MODEL), jnp.float32),
        in_specs=[pl.BlockSpec(memory_space=pltpu.MemorySpace.HBM)] * 5,
        out_specs=pl.BlockSpec(memory_space=pltpu.MemorySpace.HBM),
        scratch_shapes=[
            pltpu.VMEM((B, Sq, D_MODEL), jnp.float32),
            pltpu.VMEM((D_MODEL, D_QK), jnp.float32),
            pltpu.VMEM((B, Hq, Dh, SKV_LOC), jnp.float32),
            pltpu.VMEM((B, Hq, Dh, SKV_LOC), jnp.float32),
            pltpu.VMEM((D_QK, D_MODEL), jnp.float32),
            pltpu.VMEM((B, Sq, D_MODEL), jnp.float32),
            pltpu.VMEM((2, B, Sq, D_QK), jnp.bfloat16),
            pltpu.VMEM((2, B, Hq, 2, Sq), jnp.float32),
            pltpu.SemaphoreType.DMA((6,)),
            pltpu.SemaphoreType.DMA((3, 2)),
            pltpu.SemaphoreType.DMA((2, 2)),
            pltpu.SemaphoreType.REGULAR,
        ],
        compiler_params=pltpu.CompilerParams(collective_id=0),
    )(x, Wq, KT, VT, Wo)


# device time: 12663 ns/iter; 2.0000x vs baseline; 1.3016x over previous
import jax
import jax.numpy as jnp
from jax import lax
from jax.experimental import pallas as pl
from jax.experimental.pallas import tpu as pltpu

N_DEV = 4
B, Sq, SKV_LOC, Hq, Dh = 2, 128, 128, 4, 64
D_MODEL = 512
D_QK = Hq * Dh

EVENS = (0, 2)
SEND_TARGETS = {0: (1, 2, 3), 2: (3, 0, 1)}


def kernel(x, Wq, K_ext, V_ext, Wo):
    Q = jnp.dot(x.reshape(B * Sq, D_MODEL), Wq,
                preferred_element_type=jnp.float32)
    KT = jnp.transpose(K_ext, (0, 2, 3, 1))
    VT = jnp.transpose(V_ext, (0, 2, 3, 1))

    def body(q_ref, kt_ref, vt_ref, ctx_ref,
             o_all, ms_all, send_sems, recv_sems, done_sem):
        my = lax.axis_index("i")

        barrier = pltpu.get_barrier_semaphore()
        for tgt in EVENS:
            @pl.when(my != tgt)
            def _():
                pl.semaphore_signal(barrier, inc=1, device_id=(tgt,),
                                    device_id_type=pl.DeviceIdType.MESH)

        def compute_partial(slot, dev):
            qb = lax.broadcasted_iota(jnp.int32, (Sq, SKV_LOC), 0) // 64
            kb = lax.broadcasted_iota(jnp.int32, (Sq, SKV_LOC), 1) // 64 + 2 * dev
            mask = (qb == kb) | ((kb % 4) == (qb % 4))
            for b in range(B):
                for h in range(Hq):
                    qbh = q_ref[b * Sq:(b + 1) * Sq, h * Dh:(h + 1) * Dh]
                    s = jnp.dot(qbh, kt_ref[b, h],
                                preferred_element_type=jnp.float32) * 0.125
                    s = jnp.where(mask, s, -1e9)
                    m = jnp.max(s, axis=1)
                    w = jnp.exp(s - m[:, None])
                    ssum = jnp.sum(w, axis=1)
                    o = lax.dot_general(
                        w, vt_ref[b, h], (((1,), (1,)), ((), ())),
                        preferred_element_type=jnp.float32)
                    o_all[slot, b, :, h * Dh:(h + 1) * Dh] = o.astype(
                        jnp.bfloat16)
                    ms_all[slot, b, h, 0] = m
                    ms_all[slot, b, h, 1] = ssum

        def make_rdmas(slot, dev):
            rdmas = []
            for j, tgt in enumerate(SEND_TARGETS[dev]):
                for t, buf in enumerate((o_all, ms_all)):
                    rdmas.append(pltpu.make_async_remote_copy(
                        src_ref=buf.at[slot],
                        dst_ref=buf.at[slot],
                        send_sem=send_sems.at[j, t],
                        recv_sem=recv_sems.at[slot, t],
                        device_id=(tgt,),
                        device_id_type=pl.DeviceIdType.MESH,
                    ))
            return rdmas

        for slot, dev in enumerate(EVENS):
            @pl.when(my == dev)
            def _(slot=slot, dev=dev):
                pl.semaphore_wait(barrier, 3)
                compute_partial(slot, dev)
                for rdma in make_rdmas(slot, dev):
                    rdma.start()

        for slot, dev in enumerate(EVENS):
            @pl.when(my != dev)
            def _(slot=slot):
                for t, buf in enumerate((o_all, ms_all)):
                    recv = pltpu.make_async_remote_copy(
                        src_ref=buf.at[slot],
                        dst_ref=buf.at[slot],
                        send_sem=send_sems.at[0, t],
                        recv_sem=recv_sems.at[slot, t],
                        device_id=(0,),
                        device_id_type=pl.DeviceIdType.MESH,
                    )
                    recv.wait_recv()

        m0 = ms_all[0, :, :, 0]
        m1 = ms_all[1, :, :, 0]
        s0 = ms_all[0, :, :, 1]
        s1 = ms_all[1, :, :, 1]
        m_g = jnp.maximum(m0, m1)
        e0 = jnp.exp(m0 - m_g)
        e1 = jnp.exp(m1 - m_g)
        denom = s0 * e0 + s1 * e1
        c0 = e0 / denom
        c1 = e1 / denom

        def expand(c):
            return jnp.concatenate(
                [jnp.broadcast_to(c[:, h, :, None], (B, Sq, Dh))
                 for h in range(Hq)], axis=-1)

        ctx = (o_all[0].astype(jnp.float32) * expand(c0)
               + o_all[1].astype(jnp.float32) * expand(c1))
        ctx_ref[...] = ctx.reshape(B * Sq, D_QK)

        @pl.when((my == 1) | (my == 3))
        def _():
            for tgt in EVENS:
                pl.semaphore_signal(done_sem, inc=1, device_id=(tgt,),
                                    device_id_type=pl.DeviceIdType.MESH)

        for slot, dev in enumerate(EVENS):
            @pl.when(my == dev)
            def _(slot=slot, dev=dev):
                for rdma in make_rdmas(slot, dev):
                    rdma.wait_send()
                pl.semaphore_wait(done_sem, 2)

    ctx = pl.pallas_call(
        body,
        out_shape=jax.ShapeDtypeStruct((B * Sq, D_QK), jnp.float32),
        in_specs=[pl.BlockSpec(memory_space=pltpu.MemorySpace.VMEM)] * 3,
        out_specs=pl.BlockSpec(memory_space=pltpu.MemorySpace.VMEM),
        scratch_shapes=[
            pltpu.VMEM((2, B, Sq, D_QK), jnp.bfloat16),
            pltpu.VMEM((2, B, Hq, 2, Sq), jnp.float32),
            pltpu.SemaphoreType.DMA((3, 2)),
            pltpu.SemaphoreType.DMA((2, 2)),
            pltpu.SemaphoreType.REGULAR,
        ],
        compiler_params=pltpu.CompilerParams(collective_id=0),
    )(Q, KT, VT)

    out = jnp.dot(ctx, Wo, preferred_element_type=jnp.float32)
    return out.reshape(B, Sq, D_MODEL)


# device time: 10425 ns/iter; 2.4294x vs baseline; 1.2147x over previous
import jax
import jax.numpy as jnp
from jax import lax
from jax.experimental import pallas as pl
from jax.experimental.pallas import tpu as pltpu

N_DEV = 4
B, Sq, SKV_LOC, Hq, Dh = 2, 128, 128, 4, 64
D_MODEL = 512
D_QK = Hq * Dh

EVENS = (0, 2)
SEND_TARGETS = {0: (1, 2, 3), 2: (3, 0, 1)}


def kernel(x, Wq, K_ext, V_ext, Wo):
    Q = jnp.dot(x.reshape(B * Sq, D_MODEL), Wq,
                preferred_element_type=jnp.float32)
    KT = jnp.transpose(K_ext, (0, 2, 3, 1))
    VT = jnp.transpose(V_ext, (0, 2, 3, 1))

    def body(q_ref, kt_ref, vt_ref, ctx_ref,
             o_all, ms_all, send_sems, recv_sems):
        my = lax.axis_index("i")

        barrier = pltpu.get_barrier_semaphore()
        for tgt in EVENS:
            @pl.when(my != tgt)
            def _():
                pl.semaphore_signal(barrier, inc=1, device_id=(tgt,),
                                    device_id_type=pl.DeviceIdType.MESH)

        def compute_partial(slot, dev):
            qb = lax.broadcasted_iota(jnp.int32, (Sq, SKV_LOC), 0) // 64
            kb = lax.broadcasted_iota(jnp.int32, (Sq, SKV_LOC), 1) // 64 + 2 * dev
            mask = (qb == kb) | ((kb % 4) == (qb % 4))
            for b in range(B):
                for h in range(Hq):
                    qbh = q_ref[b * Sq:(b + 1) * Sq, h * Dh:(h + 1) * Dh]
                    s = jnp.dot(qbh, kt_ref[b, h],
                                preferred_element_type=jnp.float32) * 0.125
                    s = jnp.where(mask, s, -1e9)
                    m = jnp.max(s, axis=1)
                    w = jnp.exp(s - m[:, None])
                    ssum = jnp.sum(w, axis=1)
                    o = lax.dot_general(
                        w, vt_ref[b, h], (((1,), (1,)), ((), ())),
                        preferred_element_type=jnp.float32)
                    o_all[slot, b, :, h * Dh:(h + 1) * Dh] = o.astype(
                        jnp.bfloat16)
                    ms_all[slot, b, h, 0] = m
                    ms_all[slot, b, h, 1] = ssum

        def make_rdmas(slot, dev):
            rdmas = []
            for j, tgt in enumerate(SEND_TARGETS[dev]):
                for t, buf in enumerate((o_all, ms_all)):
                    rdmas.append(pltpu.make_async_remote_copy(
                        src_ref=buf.at[slot],
                        dst_ref=buf.at[slot],
                        send_sem=send_sems.at[j, t],
                        recv_sem=recv_sems.at[slot, t],
                        device_id=(tgt,),
                        device_id_type=pl.DeviceIdType.MESH,
                    ))
            return rdmas

        for slot, dev in enumerate(EVENS):
            @pl.when(my == dev)
            def _(slot=slot, dev=dev):
                pl.semaphore_wait(barrier, 3)
                compute_partial(slot, dev)
                for rdma in make_rdmas(slot, dev):
                    rdma.start()

        for slot, dev in enumerate(EVENS):
            @pl.when(my != dev)
            def _(slot=slot):
                for t, buf in enumerate((o_all, ms_all)):
                    recv = pltpu.make_async_remote_copy(
                        src_ref=buf.at[slot],
                        dst_ref=buf.at[slot],
                        send_sem=send_sems.at[0, t],
                        recv_sem=recv_sems.at[slot, t],
                        device_id=(0,),
                        device_id_type=pl.DeviceIdType.MESH,
                    )
                    recv.wait_recv()

        m0 = ms_all[0, :, :, 0]
        m1 = ms_all[1, :, :, 0]
        s0 = ms_all[0, :, :, 1]
        s1 = ms_all[1, :, :, 1]
        m_g = jnp.maximum(m0, m1)
        e0 = jnp.exp(m0 - m_g)
        e1 = jnp.exp(m1 - m_g)
        denom = s0 * e0 + s1 * e1
        c0 = e0 / denom
        c1 = e1 / denom

        def expand(c):
            return jnp.concatenate(
                [jnp.broadcast_to(c[:, h, :, None], (B, Sq, Dh))
                 for h in range(Hq)], axis=-1)

        ctx = (o_all[0].astype(jnp.float32) * expand(c0)
               + o_all[1].astype(jnp.float32) * expand(c1))
        ctx_ref[...] = ctx.reshape(B * Sq, D_QK)

        for slot, dev in enumerate(EVENS):
            @pl.when(my == dev)
            def _(slot=slot, dev=dev):
                for rdma in make_rdmas(slot, dev):
                    rdma.wait_send()

    ctx = pl.pallas_call(
        body,
        out_shape=jax.ShapeDtypeStruct((B * Sq, D_QK), jnp.float32),
        in_specs=[pl.BlockSpec(memory_space=pltpu.MemorySpace.VMEM)] * 3,
        out_specs=pl.BlockSpec(memory_space=pltpu.MemorySpace.VMEM),
        scratch_shapes=[
            pltpu.VMEM((2, B, Sq, D_QK), jnp.bfloat16),
            pltpu.VMEM((2, B, Hq, 2, Sq), jnp.float32),
            pltpu.SemaphoreType.DMA((3, 2)),
            pltpu.SemaphoreType.DMA((2, 2)),
        ],
        compiler_params=pltpu.CompilerParams(collective_id=0),
    )(Q, KT, VT)

    out = jnp.dot(ctx, Wo, preferred_element_type=jnp.float32)
    return out.reshape(B, Sq, D_MODEL)
